# baseline (speedup 1.0000x reference)
import functools

import jax
import jax.numpy as jnp
from jax import lax
from jax.experimental import pallas as pl
from jax.experimental.pallas import tpu as pltpu
from jax.experimental.pallas import tpu_sc as plsc

BATCH = 16384
DIM = 128
CHUNK = 128
KNOW = 128


@functools.cache
def _gather_kernel():
    info = plsc.get_sparse_core_info()
    nc, ns = info.num_cores, info.num_subcores
    nw = nc * ns
    b_per_w = BATCH // nw
    n_chunks = b_per_w // CHUNK
    mesh = plsc.VectorSubcoreMesh(core_axis_name="c", subcore_axis_name="s")

    @functools.partial(
        pl.kernel,
        mesh=mesh,
        out_type=(
            jax.ShapeDtypeStruct((BATCH, DIM), jnp.float32),
            jax.ShapeDtypeStruct((BATCH, DIM), jnp.float32),
            jax.ShapeDtypeStruct((BATCH,), jnp.float32),
            jax.ShapeDtypeStruct((KNOW, DIM), jnp.float32),
        ),
        scratch_types=[
            pltpu.VMEM((b_per_w,), jnp.int32),
            pltpu.VMEM((b_per_w,), jnp.int32),
            pltpu.VMEM((7, CHUNK, DIM), jnp.float32),
            pltpu.VMEM((b_per_w,), jnp.float32),
            pltpu.VMEM((KNOW // ns, DIM), jnp.float32),
        ] + [pltpu.SemaphoreType.DMA] * 11,
    )
    def k(sid_hbm, eid_hbm, semb_hbm, demb_hbm, disc_hbm, know_hbm,
          sout_hbm, dout_hbm, discout_hbm, knowout_hbm,
          sidx_v, eidx_v, rows_v, disc_v, know_v, *sems):
        gsems, dsem, isem_s, isem_e, ksem = (
            sems[0:7], sems[7], sems[8], sems[9], sems[10])
        wid = lax.axis_index("s") * nc + lax.axis_index("c")
        base = wid * b_per_w
        h_sidx = pltpu.async_copy(sid_hbm.at[pl.ds(base, b_per_w)], sidx_v, isem_s)
        h_eidx = pltpu.async_copy(eid_hbm.at[pl.ds(base, b_per_w)], eidx_v, isem_e)
        krows = KNOW // ns

        sizes = (32, 96, CHUNK, CHUNK, CHUNK)
        offs = (0, 32, 128, 256, 384)
        per_table = len(sizes)
        ntask = 2 * per_table
        nslot = 7

        def gather_task(t, slot):
            off, size = offs[t % per_table], sizes[t % per_table]
            sl = pl.ds(off, size)
            src = semb_hbm.at[sidx_v.at[sl]] if t < per_table \
                else demb_hbm.at[eidx_v.at[sl]]
            return pltpu.async_copy(src, rows_v.at[slot, pl.ds(0, size)],
                                    gsems[slot])

        def write_task(t, slot):
            off, size = offs[t % per_table], sizes[t % per_table]
            dst = (sout_hbm if t < per_table else dout_hbm
                   ).at[pl.ds(base + off, size)]
            return pltpu.async_copy(rows_v.at[slot, pl.ds(0, size)], dst,
                                    gsems[slot])

        g = [None] * ntask
        w = [None] * ntask
        h_sidx.wait()
        for t in range(per_table):
            g[t] = gather_task(t, t)

        @pl.when(wid < ns)
        def _():
            pltpu.async_copy(
                know_hbm.at[pl.ds(wid * krows, krows)], know_v, ksem)

        h_eidx.wait()
        disc_handles = [
            pltpu.async_copy(
                disc_hbm.at[eidx_v.at[pl.ds(j * CHUNK, CHUNK)]],
                disc_v.at[pl.ds(j * CHUNK, CHUNK)], dsem)
            for j in range(n_chunks)
        ]
        for t in range(per_table, nslot):
            g[t] = gather_task(t, t)
        for t in range(ntask):
            g[t].wait()
            w[t] = write_task(t, t % nslot)
            nt = t + nslot
            if nt < ntask:
                w[t].wait()
                g[nt] = gather_task(nt, nt % nslot)
            if t == 5:
                for h in disc_handles:
                    h.wait()
                hd = pltpu.async_copy(
                    disc_v, discout_hbm.at[pl.ds(base, b_per_w)], dsem)

                @pl.when(wid < ns)
                def _():
                    pltpu.make_async_copy(
                        know_hbm.at[pl.ds(wid * krows, krows)], know_v,
                        ksem).wait()
                    pltpu.async_copy(
                        know_v, knowout_hbm.at[pl.ds(wid * krows, krows)], ksem)

        for t in range(ntask - nslot, ntask):
            w[t].wait()
        hd.wait()

        @pl.when(wid < ns)
        def _():
            pltpu.make_async_copy(
                know_v, knowout_hbm.at[pl.ds(wid * krows, krows)], ksem).wait()

    return k


def kernel(student_id, exercise_id, q_mask, student_emb, diff_emb, disc_emb, knowledge_emb):
    del q_mask
    student_ts, diff_ts, disc_ts, knowledge_ts = _gather_kernel()(
        student_id, exercise_id, student_emb, diff_emb, disc_emb.reshape(-1),
        knowledge_emb)
    return (student_ts, diff_ts, disc_ts.reshape(-1, 1), knowledge_ts)


# speedup vs baseline: 1.0207x; 1.0207x over previous
import functools

import jax
import jax.numpy as jnp
from jax import lax
from jax.experimental import pallas as pl
from jax.experimental.pallas import tpu as pltpu
from jax.experimental.pallas import tpu_sc as plsc

BATCH = 16384
DIM = 128
CHUNK = 128
KNOW = 128


@functools.cache
def _gather_kernel():
    info = plsc.get_sparse_core_info()
    nc, ns = info.num_cores, info.num_subcores
    nw = nc * ns
    b_per_w = BATCH // nw
    n_chunks = b_per_w // CHUNK
    mesh = plsc.VectorSubcoreMesh(core_axis_name="c", subcore_axis_name="s")

    @functools.partial(
        pl.kernel,
        mesh=mesh,
        out_type=(
            jax.ShapeDtypeStruct((BATCH, DIM), jnp.float32),
            jax.ShapeDtypeStruct((BATCH, DIM), jnp.float32),
            jax.ShapeDtypeStruct((BATCH,), jnp.float32),
            jax.ShapeDtypeStruct((KNOW, DIM), jnp.float32),
        ),
        scratch_types=[
            pltpu.VMEM((b_per_w,), jnp.int32),
            pltpu.VMEM((b_per_w,), jnp.int32),
            pltpu.VMEM((7, CHUNK, DIM), jnp.float32),
            pltpu.VMEM((b_per_w,), jnp.float32),
            pltpu.VMEM((KNOW // ns, DIM), jnp.float32),
        ] + [pltpu.SemaphoreType.DMA] * 11,
    )
    def k(sid_hbm, eid_hbm, semb_hbm, demb_hbm, disc_hbm, know_hbm,
          sout_hbm, dout_hbm, discout_hbm, knowout_hbm,
          sidx_v, eidx_v, rows_v, disc_v, know_v, *sems):
        gsems, dsem, isem_s, isem_e, ksem = (
            sems[0:7], sems[7], sems[8], sems[9], sems[10])
        wid = lax.axis_index("s") * nc + lax.axis_index("c")
        base = wid * b_per_w
        h_sidx = pltpu.async_copy(sid_hbm.at[pl.ds(base, b_per_w)], sidx_v, isem_s)
        h_eidx = pltpu.async_copy(eid_hbm.at[pl.ds(base, b_per_w)], eidx_v, isem_e)
        krows = KNOW // ns

        ntask = 2 * n_chunks
        nslot = 7

        def gather_task(t, slot):
            j = t % n_chunks
            sl = pl.ds(j * CHUNK, CHUNK)
            src = semb_hbm.at[sidx_v.at[sl]] if t < n_chunks \
                else demb_hbm.at[eidx_v.at[sl]]
            return pltpu.async_copy(src, rows_v.at[slot], gsems[slot])

        def write_task(t, slot):
            j = t % n_chunks
            dst = (sout_hbm if t < n_chunks else dout_hbm
                   ).at[pl.ds(base + j * CHUNK, CHUNK)]
            return pltpu.async_copy(rows_v.at[slot], dst, gsems[slot])

        g = [None] * ntask
        w = [None] * ntask
        h_sidx.wait()
        for t in range(n_chunks):
            g[t] = gather_task(t, t)

        @pl.when(wid < ns)
        def _():
            pltpu.async_copy(
                know_hbm.at[pl.ds(wid * krows, krows)], know_v, ksem)

        h_eidx.wait()
        for t in range(n_chunks, nslot):
            g[t] = gather_task(t, t)
        disc_handles = [
            pltpu.async_copy(
                disc_hbm.at[eidx_v.at[pl.ds(j * CHUNK, CHUNK)]],
                disc_v.at[pl.ds(j * CHUNK, CHUNK)], dsem)
            for j in range(n_chunks)
        ]
        for t in range(ntask):
            g[t].wait()
            w[t] = write_task(t, t % nslot)
            nt = t + nslot
            if nt < ntask:
                w[t].wait()
                g[nt] = gather_task(nt, nt % nslot)
            if t == 4:
                for h in disc_handles:
                    h.wait()
                hd = pltpu.async_copy(
                    disc_v, discout_hbm.at[pl.ds(base, b_per_w)], dsem)

                @pl.when(wid < ns)
                def _():
                    pltpu.make_async_copy(
                        know_hbm.at[pl.ds(wid * krows, krows)], know_v,
                        ksem).wait()
                    pltpu.async_copy(
                        know_v, knowout_hbm.at[pl.ds(wid * krows, krows)], ksem)

        for t in range(ntask - nslot, ntask):
            w[t].wait()
        hd.wait()

        @pl.when(wid < ns)
        def _():
            pltpu.make_async_copy(
                know_v, knowout_hbm.at[pl.ds(wid * krows, krows)], ksem).wait()

    return k


def kernel(student_id, exercise_id, q_mask, student_emb, diff_emb, disc_emb, knowledge_emb):
    del q_mask
    student_ts, diff_ts, disc_ts, knowledge_ts = _gather_kernel()(
        student_id, exercise_id, student_emb, diff_emb, disc_emb.reshape(-1),
        knowledge_emb)
    return (student_ts, diff_ts, disc_ts.reshape(-1, 1), knowledge_ts)
